# Initial kernel scaffold; baseline (speedup 1.0000x reference)
import functools

import jax
import jax.numpy as jnp
from jax import lax
from jax.experimental import pallas as pl
from jax.experimental.pallas import tpu as pltpu
from jax.experimental.pallas import tpu_sc as plsc

N = 10000
E = 160000
D_IN = 256
D_HID = 512
D_CLS = 128

C = 128
EP = 172032
NCHUNK = EP // C
NTILES = 16
CHUNKS_PER_TILE = NCHUNK // NTILES
RA = 10016
ZROWS = RA // NTILES
WROWS = N // NTILES

_mesh = plsc.VectorSubcoreMesh(core_axis_name="c", subcore_axis_name="s")


def _fill_const(ref, rows, width, value):
    @pl.loop(0, rows)
    def _(r):
        @pl.loop(0, width, step=16)
        def _(c):
            ref[r, pl.ds(c, 16)] = jnp.full((16,), value, jnp.float32)


DW = 16


@functools.partial(
    pl.kernel,
    out_type=jax.ShapeDtypeStruct((RA, DW), jnp.float32),
    mesh=_mesh,
    scratch_types=[
        pltpu.VMEM((2, C), jnp.int32),
        pltpu.VMEM((C, DW), jnp.float32),
        pltpu.VMEM((128, DW), jnp.float32),
        pltpu.VMEM_SHARED((RA, DW), jnp.float32),
        pltpu.SemaphoreType.DMA,
    ],
)
def _deg_kernel(dstp_hbm, out_hbm, dstv, ones, zbuf, acc, sem):
    cid = lax.axis_index("c")
    sid = lax.axis_index("s")

    @pl.when(cid == 0)
    def _():
        _fill_const(ones, C, DW, 1.0)
        _fill_const(zbuf, 128, DW, 0.0)
        zb = sid * ZROWS
        for off in range(0, ZROWS, 128):
            n = min(128, ZROWS - off)
            pltpu.sync_copy(zbuf.at[pl.ds(0, n)], acc.at[pl.ds(zb + off, n)])
        plsc.subcore_barrier()

        @pl.loop(0, CHUNKS_PER_TILE)
        def _(k):
            eb = (sid * CHUNKS_PER_TILE + k) * C
            pltpu.sync_copy(dstp_hbm.at[pl.ds(eb, C)], dstv.at[0])
            pltpu.sync_copy(ones, acc.at[dstv.at[0]], add=True)

        plsc.subcore_barrier()
        pltpu.sync_copy(acc.at[pl.ds(sid * ZROWS, ZROWS)],
                        out_hbm.at[pl.ds(sid * ZROWS, ZROWS)])


def _make_prop(S, W):
    S_per_core = S // 2

    @functools.partial(
        pl.kernel,
        out_type=jax.ShapeDtypeStruct((S, N, W), jnp.float32),
        mesh=_mesh,
        scratch_types=[
            pltpu.VMEM((2, C), jnp.int32),
            pltpu.VMEM((2, C), jnp.int32),
            pltpu.VMEM((C, W), jnp.float32),
            pltpu.VMEM((128, W), jnp.float32),
            pltpu.VMEM_SHARED((RA, W), jnp.float32),
            pltpu.SemaphoreType.DMA,
        ],
    )
    def _prop(xws_hbm, srcp_hbm, dstp_hbm, out_hbm, srcv, dstv, rows, zbuf,
              acc, sem):
        cid = lax.axis_index("c")
        sid = lax.axis_index("s")
        _fill_const(zbuf, 128, W, 0.0)

        for j in range(S_per_core):
            slice_id = cid * S_per_core + j
            zb = sid * ZROWS
            for off in range(0, ZROWS, 128):
                n = min(128, ZROWS - off)
                pltpu.sync_copy(zbuf.at[pl.ds(0, n)], acc.at[pl.ds(zb + off, n)])
            plsc.subcore_barrier()

            @pl.loop(0, CHUNKS_PER_TILE)
            def _(k):
                eb = (sid * CHUNKS_PER_TILE + k) * C
                pltpu.sync_copy(srcp_hbm.at[pl.ds(eb, C)], srcv.at[0])
                pltpu.sync_copy(dstp_hbm.at[pl.ds(eb, C)], dstv.at[0])
                pltpu.async_copy(xws_hbm.at[slice_id].at[srcv.at[0]], rows,
                                 sem).wait()
                pltpu.sync_copy(rows, acc.at[dstv.at[0]], add=True)

            plsc.subcore_barrier()
            wb = sid * WROWS
            pltpu.sync_copy(acc.at[pl.ds(wb, WROWS)],
                            out_hbm.at[slice_id].at[pl.ds(wb, WROWS)])
            plsc.subcore_barrier()

    return _prop


_prop_l1 = _make_prop(4, 128)
_prop_l2 = _make_prop(2, 64)


def _dinv_of(deg_blk):
    return lax.rsqrt(jnp.maximum(deg_blk[:, 0:1], 1e-12))


def _mm1_body(x_ref, w_ref, deg_ref, out_ref):
    acc = jnp.dot(x_ref[...], w_ref[...], preferred_element_type=jnp.float32)
    acc = acc * _dinv_of(deg_ref[...])
    for s in range(4):
        out_ref[s] = acc[:, s * 128:(s + 1) * 128]


def _mm2_body(h1_ref, w2_ref, deg_ref, out_ref):
    dinv = _dinv_of(deg_ref[...])
    acc = jnp.zeros((h1_ref.shape[1], 128), jnp.float32)
    for s in range(4):
        h = jnp.maximum(h1_ref[s] * dinv, 0.0)
        acc = acc + jnp.dot(h, w2_ref[s], preferred_element_type=jnp.float32)
    acc = acc * dinv
    out_ref[0] = acc[:, :64]
    out_ref[1] = acc[:, 64:]


def _final_body(h2_ref, deg_ref, h3_ref, sm_ref):
    dinv = _dinv_of(deg_ref[...])
    h3 = jnp.concatenate([h2_ref[0], h2_ref[1]], axis=1) * dinv
    m = jnp.max(h3, axis=1, keepdims=True)
    e = jnp.exp(h3 - m)
    h3_ref[...] = h3
    sm_ref[...] = e / jnp.sum(e, axis=1, keepdims=True)


BN = 1000


def _mm1(x, W1, deg):
    return pl.pallas_call(
        _mm1_body,
        grid=(N // BN,),
        in_specs=[
            pl.BlockSpec((BN, D_IN), lambda i: (i, 0)),
            pl.BlockSpec((D_IN, D_HID), lambda i: (0, 0)),
            pl.BlockSpec((BN, DW), lambda i: (i, 0)),
        ],
        out_specs=pl.BlockSpec((4, BN, 128), lambda i: (0, i, 0)),
        out_shape=jax.ShapeDtypeStruct((4, N, 128), jnp.float32),
    )(x, W1, deg)


def _mm2(h1raw, W2r, deg):
    return pl.pallas_call(
        _mm2_body,
        grid=(N // BN,),
        in_specs=[
            pl.BlockSpec((4, BN, 128), lambda i: (0, i, 0)),
            pl.BlockSpec((4, 128, 128), lambda i: (0, 0, 0)),
            pl.BlockSpec((BN, DW), lambda i: (i, 0)),
        ],
        out_specs=pl.BlockSpec((2, BN, 64), lambda i: (0, i, 0)),
        out_shape=jax.ShapeDtypeStruct((2, N, 64), jnp.float32),
    )(h1raw, W2r, deg)


def _final(h2raw, deg):
    return pl.pallas_call(
        _final_body,
        grid=(N // BN,),
        in_specs=[
            pl.BlockSpec((2, BN, 64), lambda i: (0, i, 0)),
            pl.BlockSpec((BN, DW), lambda i: (i, 0)),
        ],
        out_specs=(
            pl.BlockSpec((BN, D_CLS), lambda i: (i, 0)),
            pl.BlockSpec((BN, D_CLS), lambda i: (i, 0)),
        ),
        out_shape=(
            jax.ShapeDtypeStruct((N, D_CLS), jnp.float32),
            jax.ShapeDtypeStruct((N, D_CLS), jnp.float32),
        ),
    )(h2raw, deg)


def kernel(x, edge_index, batch_index, W1, W2):
    loop = jnp.arange(N, dtype=jnp.int32)
    npad = EP - E - N
    srcp = jnp.concatenate([edge_index[0], loop,
                            jnp.zeros((npad,), jnp.int32)])
    dstp = jnp.concatenate([edge_index[1], loop,
                            jnp.full((npad,), N, jnp.int32)])

    deg = _deg_kernel(dstp)
    xw1s = _mm1(x, W1, deg)
    h1raw = _prop_l1(xw1s, srcp, dstp)
    xw2s = _mm2(h1raw, W2.reshape(4, 128, D_CLS), deg)
    h2raw = _prop_l2(xw2s, srcp, dstp)
    hidden3, output = _final(h2raw, deg)
    return (hidden3, output)


# speedup vs baseline: 6.2835x; 6.2835x over previous
import functools

import jax
import jax.numpy as jnp
from jax import lax
from jax.experimental import pallas as pl
from jax.experimental.pallas import tpu as pltpu
from jax.experimental.pallas import tpu_sc as plsc

N = 10000
E = 160000
D_IN = 256
D_HID = 512
D_CLS = 128

C = 128
EP = 172032
NCHUNK = EP // C
NTILES = 16
CHUNKS_PER_TILE = NCHUNK // NTILES
RA = 10240
ZROWS = RA // NTILES
WROWS = RA // NTILES

_mesh = plsc.VectorSubcoreMesh(core_axis_name="c", subcore_axis_name="s")


def _fill_const(ref, rows, width, value):
    @pl.loop(0, rows)
    def _(r):
        @pl.loop(0, width, step=16)
        def _(c):
            ref[r, pl.ds(c, 16)] = jnp.full((16,), value, jnp.float32)


DW = 128


@functools.partial(
    pl.kernel,
    out_type=jax.ShapeDtypeStruct((RA, DW), jnp.float32),

    mesh=_mesh,
    scratch_types=[
        pltpu.VMEM((2, C), jnp.int32),
        pltpu.VMEM((C, DW), jnp.float32),
        pltpu.VMEM((128, DW), jnp.float32),
        pltpu.VMEM_SHARED((RA, DW), jnp.float32),
        pltpu.SemaphoreType.DMA,
    ],
)
def _deg_kernel(dstp_hbm, out_hbm, dstv, ones, zbuf, acc, sem):
    cid = lax.axis_index("c")
    sid = lax.axis_index("s")

    @pl.when(cid == 0)
    def _():
        _fill_const(ones, C, DW, 1.0)
        _fill_const(zbuf, 128, DW, 0.0)
        zb = sid * ZROWS
        for off in range(0, ZROWS, 128):
            n = min(128, ZROWS - off)
            pltpu.sync_copy(zbuf.at[pl.ds(0, n)], acc.at[pl.ds(zb + off, n)])
        plsc.subcore_barrier()

        @pl.loop(0, CHUNKS_PER_TILE)
        def _(k):
            eb = (sid * CHUNKS_PER_TILE + k) * C
            pltpu.sync_copy(dstp_hbm.at[pl.ds(eb, C)], dstv.at[0])
            pltpu.sync_copy(ones, acc.at[dstv.at[0]], add=True)

        plsc.subcore_barrier()
        pltpu.sync_copy(acc.at[pl.ds(sid * ZROWS, ZROWS)],
                        out_hbm.at[pl.ds(sid * ZROWS, ZROWS)])


def _make_prop(S, W):
    S_per_core = S // 2

    @functools.partial(
        pl.kernel,
        out_type=jax.ShapeDtypeStruct((S, RA, W), jnp.float32),
        mesh=_mesh,
        scratch_types=[
            pltpu.VMEM((2, C), jnp.int32),
            pltpu.VMEM((2, C), jnp.int32),
            pltpu.VMEM((C, W), jnp.float32),
            pltpu.VMEM((128, W), jnp.float32),
            pltpu.VMEM_SHARED((RA, W), jnp.float32),
            pltpu.SemaphoreType.DMA,
        ],
    )
    def _prop(xws_hbm, srcp_hbm, dstp_hbm, out_hbm, srcv, dstv, rows, zbuf,
              acc, sem):
        cid = lax.axis_index("c")
        sid = lax.axis_index("s")
        _fill_const(zbuf, 128, W, 0.0)

        for j in range(S_per_core):
            slice_id = cid * S_per_core + j
            zb = sid * ZROWS
            for off in range(0, ZROWS, 128):
                n = min(128, ZROWS - off)
                pltpu.sync_copy(zbuf.at[pl.ds(0, n)], acc.at[pl.ds(zb + off, n)])
            plsc.subcore_barrier()

            @pl.loop(0, CHUNKS_PER_TILE)
            def _(k):
                eb = (sid * CHUNKS_PER_TILE + k) * C
                pltpu.sync_copy(srcp_hbm.at[pl.ds(eb, C)], srcv.at[0])
                pltpu.sync_copy(dstp_hbm.at[pl.ds(eb, C)], dstv.at[0])
                pltpu.async_copy(xws_hbm.at[slice_id].at[srcv.at[0]], rows,
                                 sem).wait()
                pltpu.sync_copy(rows, acc.at[dstv.at[0]], add=True)

            plsc.subcore_barrier()
            wb = sid * WROWS
            pltpu.sync_copy(acc.at[pl.ds(wb, WROWS)],
                            out_hbm.at[slice_id].at[pl.ds(wb, WROWS)])
            plsc.subcore_barrier()

    return _prop


_prop_l1 = _make_prop(4, 128)


@functools.partial(
    pl.kernel,
    out_type=jax.ShapeDtypeStruct((2, RA, D_CLS), jnp.float32),
    mesh=_mesh,
    scratch_types=[
        pltpu.VMEM((2, C), jnp.int32),
        pltpu.VMEM((2, C), jnp.int32),
        pltpu.VMEM((C, D_CLS), jnp.float32),
        pltpu.VMEM((128, D_CLS), jnp.float32),
        pltpu.VMEM_SHARED((RA, D_CLS), jnp.float32),
        pltpu.SemaphoreType.DMA,
    ],
)
def _prop_l2(xws_hbm, srcp_hbm, dstp_hbm, out_hbm, srcv, dstv, rows, zbuf,
             acc, sem):
    cid = lax.axis_index("c")
    sid = lax.axis_index("s")
    _fill_const(zbuf, 128, D_CLS, 0.0)
    zb = sid * ZROWS
    for off in range(0, ZROWS, 128):
        n = min(128, ZROWS - off)
        pltpu.sync_copy(zbuf.at[pl.ds(0, n)], acc.at[pl.ds(zb + off, n)])
    plsc.subcore_barrier()

    half = NCHUNK // 2

    @pl.loop(0, half // NTILES)
    def _(k):
        eb = (cid * half + sid * (half // NTILES) + k) * C
        pltpu.sync_copy(srcp_hbm.at[pl.ds(eb, C)], srcv.at[0])
        pltpu.sync_copy(dstp_hbm.at[pl.ds(eb, C)], dstv.at[0])
        pltpu.async_copy(xws_hbm.at[srcv.at[0]], rows, sem).wait()
        pltpu.sync_copy(rows, acc.at[dstv.at[0]], add=True)

    plsc.subcore_barrier()
    wb = sid * WROWS
    pltpu.sync_copy(acc.at[pl.ds(wb, WROWS)],
                    out_hbm.at[cid].at[pl.ds(wb, WROWS)])


def _dinv_of(deg_blk):
    return lax.rsqrt(jnp.maximum(deg_blk[:, 0:1], 1e-12))


def _mm1_body(x_ref, w_ref, deg_ref, out_ref):
    acc = jnp.dot(x_ref[...], w_ref[...], preferred_element_type=jnp.float32)
    acc = acc * _dinv_of(deg_ref[...])
    for s in range(4):
        out_ref[s] = acc[:, s * 128:(s + 1) * 128]


def _mm2_body(h1_ref, w2_ref, deg_ref, out_ref):
    dinv = _dinv_of(deg_ref[...])
    acc = jnp.zeros((h1_ref.shape[1], 128), jnp.float32)
    for s in range(4):
        h = jnp.maximum(h1_ref[s] * dinv, 0.0)
        acc = acc + jnp.dot(h, w2_ref[s], preferred_element_type=jnp.float32)
    acc = acc * dinv
    out_ref[...] = acc


def _final_body(h2_ref, deg_ref, h3_ref, sm_ref):
    dinv = _dinv_of(deg_ref[...])
    h3 = (h2_ref[0] + h2_ref[1]) * dinv
    m = jnp.max(h3, axis=1, keepdims=True)
    e = jnp.exp(h3 - m)
    h3_ref[...] = h3
    sm_ref[...] = e / jnp.sum(e, axis=1, keepdims=True)


BN = 1000


def _mm1(x, W1, deg):
    return pl.pallas_call(
        _mm1_body,
        grid=(N // BN,),
        in_specs=[
            pl.BlockSpec((BN, D_IN), lambda i: (i, 0)),
            pl.BlockSpec((D_IN, D_HID), lambda i: (0, 0)),
            pl.BlockSpec((BN, DW), lambda i: (i, 0)),
        ],
        out_specs=pl.BlockSpec((4, BN, 128), lambda i: (0, i, 0)),
        out_shape=jax.ShapeDtypeStruct((4, N, 128), jnp.float32),
    )(x, W1, deg)


def _mm2(h1raw, W2r, deg):
    return pl.pallas_call(
        _mm2_body,
        grid=(N // BN,),
        in_specs=[
            pl.BlockSpec((4, BN, 128), lambda i: (0, i, 0)),
            pl.BlockSpec((4, 128, 128), lambda i: (0, 0, 0)),
            pl.BlockSpec((BN, DW), lambda i: (i, 0)),
        ],
        out_specs=pl.BlockSpec((BN, D_CLS), lambda i: (i, 0)),
        out_shape=jax.ShapeDtypeStruct((N, D_CLS), jnp.float32),
    )(h1raw, W2r, deg)


def _final(h2raw, deg):
    return pl.pallas_call(
        _final_body,
        grid=(N // BN,),
        in_specs=[
            pl.BlockSpec((2, BN, D_CLS), lambda i: (0, i, 0)),
            pl.BlockSpec((BN, DW), lambda i: (i, 0)),
        ],
        out_specs=(
            pl.BlockSpec((BN, D_CLS), lambda i: (i, 0)),
            pl.BlockSpec((BN, D_CLS), lambda i: (i, 0)),
        ),
        out_shape=(
            jax.ShapeDtypeStruct((N, D_CLS), jnp.float32),
            jax.ShapeDtypeStruct((N, D_CLS), jnp.float32),
        ),
    )(h2raw, deg)


def kernel(x, edge_index, batch_index, W1, W2):
    loop = jnp.arange(N, dtype=jnp.int32)
    npad = EP - E - N
    srcp = jnp.concatenate([edge_index[0], loop,
                            jnp.zeros((npad,), jnp.int32)])
    dstp = jnp.concatenate([edge_index[1], loop,
                            jnp.full((npad,), N, jnp.int32)])

    deg = _deg_kernel(dstp)
    xw1s = _mm1(x, W1, deg)
    h1raw = _prop_l1(xw1s, srcp, dstp)
    xw2s = _mm2(h1raw, W2.reshape(4, 128, D_CLS), deg)
    h2raw = _prop_l2(xw2s, srcp, dstp)
    hidden3, output = _final(h2raw, deg)
    return (hidden3, output)
